# Initial kernel scaffold; baseline (speedup 1.0000x reference)
import functools

import jax
import jax.numpy as jnp
from jax import lax
from jax.experimental import pallas as pl
from jax.experimental.pallas import tpu as pltpu
from jax.experimental.pallas import tpu_sc as plsc

N_NODES = 10000
D = 256
HALF = 128
N_EDGES = 160000
EPS = 1e-5

NC = 2
NS = 16
L = 16

CHUNK = 128
EPT = 10240
E_PAD = EPT * NS
NCHUNKS = EPT // CHUNK
ACC_ROWS = 10240
ZPT = ACC_ROWS // NS
RPT = N_NODES // NS
DEG_EPW = E_PAD // (NC * NS)

ROWBLK = 1000
GRID = N_NODES // ROWBLK

_mesh = plsc.VectorSubcoreMesh(core_axis_name="c", subcore_axis_name="s")


@functools.partial(
    pl.kernel,
    out_type=jax.ShapeDtypeStruct((NC * NS, ACC_ROWS), jnp.float32),
    mesh=_mesh,
    scratch_types=[
        pltpu.VMEM((DEG_EPW,), jnp.int32),
        pltpu.VMEM((ACC_ROWS,), jnp.float32),
    ],
)
def _deg_kernel(dst_hbm, degp_hbm, dbuf, table):
    c = lax.axis_index("c")
    s = lax.axis_index("s")
    w = c * NS + s

    zv = jnp.zeros((L,), jnp.float32)

    def zbody(i, _):
        table[pl.ds(i * L, L)] = zv
        return 0

    lax.fori_loop(0, ACC_ROWS // L, zbody, 0)

    pltpu.sync_copy(dst_hbm.at[pl.ds(w * DEG_EPW, DEG_EPW)], dbuf)

    ones = jnp.ones((L,), jnp.float32)

    def body(i, _):
        d = dbuf[pl.ds(i * L, L)]
        plsc.addupdate_scatter(table, [d], ones)
        return 0

    lax.fori_loop(0, DEG_EPW // L, body, 0)

    pltpu.sync_copy(table, degp_hbm.at[w])


@functools.partial(
    pl.kernel,
    out_type=jax.ShapeDtypeStruct((NC, N_NODES, HALF), jnp.float32),
    mesh=_mesh,
    scratch_types=[
        pltpu.VMEM((CHUNK,), jnp.int32),
        pltpu.VMEM((CHUNK,), jnp.int32),
        pltpu.VMEM((CHUNK, HALF), jnp.float32),
        pltpu.VMEM_SHARED((ACC_ROWS, HALF), jnp.float32),
        pltpu.SemaphoreType.DMA,
    ],
)
def _prop_kernel(hs_hbm, src2_hbm, dst_hbm, out_hbm, sbuf, dbuf, rows, acc, sem):
    c = lax.axis_index("c")
    s = lax.axis_index("s")

    zv = jnp.zeros((L,), jnp.float32)

    def zrows(i, _):
        r = i // (HALF // L)
        j = i % (HALF // L)
        rows[r, pl.ds(j * L, L)] = zv
        return 0

    lax.fori_loop(0, CHUNK * (HALF // L), zrows, 0)

    def zacc(k, _):
        pltpu.sync_copy(rows, acc.at[pl.ds(s * ZPT + k * CHUNK, CHUNK)])
        return 0

    lax.fori_loop(0, ZPT // CHUNK, zacc, 0)
    plsc.subcore_barrier()

    def chunk_body(g, _):
        eb = s * EPT + g * CHUNK
        pltpu.sync_copy(src2_hbm.at[c, pl.ds(eb, CHUNK)], sbuf)
        pltpu.sync_copy(dst_hbm.at[pl.ds(eb, CHUNK)], dbuf)
        pltpu.async_copy(hs_hbm.at[sbuf], rows, sem).wait()
        pltpu.sync_copy(rows, acc.at[dbuf], add=True)
        return 0

    lax.fori_loop(0, NCHUNKS, chunk_body, 0)
    plsc.subcore_barrier()

    pltpu.sync_copy(acc.at[pl.ds(s * RPT, RPT)], out_hbm.at[c, pl.ds(s * RPT, RPT)])


def _dinv_from_partials(degp_blk):
    deg = jnp.sum(degp_blk, axis=0) + 1.0
    return lax.rsqrt(deg)


def _mm1_body(x_ref, w_ref, degp_ref, out_ref):
    dinv = _dinv_from_partials(degp_ref[...])
    h = jnp.dot(x_ref[...], w_ref[...], preferred_element_type=jnp.float32)
    hp = h * dinv[:, None]
    out_ref[0] = hp[:, :HALF]
    out_ref[1] = hp[:, HALF:]


def _mm1(x, w1, degp):
    return pl.pallas_call(
        _mm1_body,
        grid=(GRID,),
        in_specs=[
            pl.BlockSpec((ROWBLK, D), lambda i: (i, 0)),
            pl.BlockSpec((D, D), lambda i: (0, 0)),
            pl.BlockSpec((NC * NS, ROWBLK), lambda i: (0, i)),
        ],
        out_specs=pl.BlockSpec((NC, ROWBLK, HALF), lambda i: (0, i, 0)),
        out_shape=jax.ShapeDtypeStruct((NC, N_NODES, HALF), jnp.float32),
    )(x, w1, degp)


def _zstats_body(acc_ref, hs_ref, degp_ref, b_ref, z_ref, stats_ref, s0, s1):
    i = pl.program_id(0)
    dinv = _dinv_from_partials(degp_ref[...])
    accv = jnp.concatenate([acc_ref[0], acc_ref[1]], axis=1)
    hp = jnp.concatenate([hs_ref[0], hs_ref[1]], axis=1)
    z = dinv[:, None] * (accv + hp) + b_ref[...]
    z_ref[...] = z

    @pl.when(i == 0)
    def _():
        s0[...] = jnp.zeros_like(s0)
        s1[...] = jnp.zeros_like(s1)

    s0[...] += jnp.sum(z, axis=0, keepdims=True)
    s1[...] += jnp.sum(z * z, axis=0, keepdims=True)

    @pl.when(i == GRID - 1)
    def _():
        stats_ref[...] = jnp.concatenate([s0[...], s1[...]], axis=0)


def _zstats(acc1, hs1, degp, b1):
    return pl.pallas_call(
        _zstats_body,
        grid=(GRID,),
        in_specs=[
            pl.BlockSpec((NC, ROWBLK, HALF), lambda i: (0, i, 0)),
            pl.BlockSpec((NC, ROWBLK, HALF), lambda i: (0, i, 0)),
            pl.BlockSpec((NC * NS, ROWBLK), lambda i: (0, i)),
            pl.BlockSpec((1, D), lambda i: (0, 0)),
        ],
        out_specs=[
            pl.BlockSpec((ROWBLK, D), lambda i: (i, 0)),
            pl.BlockSpec((2, D), lambda i: (0, 0)),
        ],
        out_shape=[
            jax.ShapeDtypeStruct((N_NODES, D), jnp.float32),
            jax.ShapeDtypeStruct((2, D), jnp.float32),
        ],
        scratch_shapes=[
            pltpu.VMEM((1, D), jnp.float32),
            pltpu.VMEM((1, D), jnp.float32),
        ],
    )(acc1, hs1, degp, b1)


def _layer2_body(z_ref, stats_ref, g_ref, be_ref, w_ref, degp_ref, out_ref):
    mean = stats_ref[0:1, :] * (1.0 / N_NODES)
    ex2 = stats_ref[1:2, :] * (1.0 / N_NODES)
    var = ex2 - mean * mean
    xhat = (z_ref[...] - mean) * lax.rsqrt(var + EPS)
    y = jnp.maximum(xhat * g_ref[...] + be_ref[...], 0.0)
    h2 = jnp.dot(y, w_ref[...], preferred_element_type=jnp.float32)
    dinv = _dinv_from_partials(degp_ref[...])
    hp2 = h2 * dinv[:, None]
    out_ref[0] = hp2[:, :HALF]
    out_ref[1] = hp2[:, HALF:]


def _layer2(z1, stats, gamma, beta, w2, degp):
    return pl.pallas_call(
        _layer2_body,
        grid=(GRID,),
        in_specs=[
            pl.BlockSpec((ROWBLK, D), lambda i: (i, 0)),
            pl.BlockSpec((2, D), lambda i: (0, 0)),
            pl.BlockSpec((1, D), lambda i: (0, 0)),
            pl.BlockSpec((1, D), lambda i: (0, 0)),
            pl.BlockSpec((D, D), lambda i: (0, 0)),
            pl.BlockSpec((NC * NS, ROWBLK), lambda i: (0, i)),
        ],
        out_specs=pl.BlockSpec((NC, ROWBLK, HALF), lambda i: (0, i, 0)),
        out_shape=jax.ShapeDtypeStruct((NC, N_NODES, HALF), jnp.float32),
    )(z1, stats, gamma, beta, w2, degp)


def _final_body(acc_ref, hs_ref, degp_ref, b_ref, o_ref):
    dinv = _dinv_from_partials(degp_ref[...])
    accv = jnp.concatenate([acc_ref[0], acc_ref[1]], axis=1)
    hp = jnp.concatenate([hs_ref[0], hs_ref[1]], axis=1)
    o_ref[...] = dinv[:, None] * (accv + hp) + b_ref[...]


def _final(acc2, hs2, degp, b2):
    return pl.pallas_call(
        _final_body,
        grid=(GRID,),
        in_specs=[
            pl.BlockSpec((NC, ROWBLK, HALF), lambda i: (0, i, 0)),
            pl.BlockSpec((NC, ROWBLK, HALF), lambda i: (0, i, 0)),
            pl.BlockSpec((NC * NS, ROWBLK), lambda i: (0, i)),
            pl.BlockSpec((1, D), lambda i: (0, 0)),
        ],
        out_specs=pl.BlockSpec((ROWBLK, D), lambda i: (i, 0)),
        out_shape=jax.ShapeDtypeStruct((N_NODES, D), jnp.float32),
    )(acc2, hs2, degp, b2)


def kernel(x, edge_index, W1, b1, gamma, beta, W2, b2):
    ei = edge_index.astype(jnp.int32)
    src = ei[0]
    dst = ei[1]
    npad = E_PAD - N_EDGES
    src_p = jnp.concatenate([src, jnp.zeros((npad,), jnp.int32)])
    dst_p = jnp.concatenate([dst, jnp.full((npad,), N_NODES, jnp.int32)])
    src2 = jnp.stack([src_p, src_p + N_NODES])

    b1r = b1.reshape(1, D)
    b2r = b2.reshape(1, D)
    gammar = gamma.reshape(1, D)
    betar = beta.reshape(1, D)

    degp = _deg_kernel(dst_p)
    hs1 = _mm1(x, W1, degp)
    acc1 = _prop_kernel(hs1.reshape(NC * N_NODES, HALF), src2, dst_p)
    z1, stats = _zstats(acc1, hs1, degp, b1r)
    hs2 = _layer2(z1, stats, gammar, betar, W2, degp)
    acc2 = _prop_kernel(hs2.reshape(NC * N_NODES, HALF), src2, dst_p)
    return _final(acc2, hs2, degp, b2r)


# speedup vs baseline: 6.1959x; 6.1959x over previous
import functools

import jax
import jax.numpy as jnp
from jax import lax
from jax.experimental import pallas as pl
from jax.experimental.pallas import tpu as pltpu
from jax.experimental.pallas import tpu_sc as plsc

N_NODES = 10000
D = 256
HALF = 128
N_EDGES = 160000
EPS = 1e-5

NC = 2
NS = 16
L = 16

CHUNK = 128
EPT = 10240
E_PAD = EPT * NS
NCHUNKS = EPT // CHUNK
ACC_ROWS = 10240
ZPT = ACC_ROWS // NS
RPT = N_NODES // NS
DEG_EPW = E_PAD // (NC * NS)

ROWBLK = 1000
GRID = N_NODES // ROWBLK

DEG_W = 16
DEG_CH = DEG_EPW // CHUNK


@functools.cache
def _build_deg_kernel():
  mesh = plsc.VectorSubcoreMesh(core_axis_name="c", subcore_axis_name="s")

  @functools.partial(
      pl.kernel,
      out_type=jax.ShapeDtypeStruct((NC, ACC_ROWS, DEG_W), jnp.float32),
      mesh=mesh,
      scratch_types=[
          pltpu.VMEM((CHUNK,), jnp.int32),
          pltpu.VMEM((CHUNK, DEG_W), jnp.float32),
          pltpu.VMEM((CHUNK, DEG_W), jnp.float32),
          pltpu.VMEM_SHARED((ACC_ROWS, DEG_W), jnp.float32),
      ],
  )
  def deg_kernel(dst_hbm, degp_hbm, dbuf, ones, zbuf, table):
    c = lax.axis_index("c")
    s = lax.axis_index("s")

    onev = jnp.ones((L,), jnp.float32)
    zv = jnp.zeros((L,), jnp.float32)

    def fbody(i, _):
        ones[i, pl.ds(0, L)] = onev
        zbuf[i, pl.ds(0, L)] = zv
        return 0

    lax.fori_loop(0, CHUNK, fbody, 0)

    def zacc(k, _):
        pltpu.sync_copy(zbuf, table.at[pl.ds(s * ZPT + k * CHUNK, CHUNK)])
        return 0

    lax.fori_loop(0, ZPT // CHUNK, zacc, 0)
    plsc.subcore_barrier()

    def body(g, _):
        eb = (c * NS + s) * DEG_EPW + g * CHUNK
        pltpu.sync_copy(dst_hbm.at[pl.ds(eb, CHUNK)], dbuf)
        pltpu.sync_copy(ones, table.at[dbuf], add=True)
        return 0

    lax.fori_loop(0, DEG_CH, body, 0)
    plsc.subcore_barrier()

    pltpu.sync_copy(
        table.at[pl.ds(s * ZPT, ZPT)], degp_hbm.at[c, pl.ds(s * ZPT, ZPT)]
    )

  return deg_kernel


def _deg_kernel(dst_p):
  return _build_deg_kernel()(dst_p)


@functools.cache
def _build_prop_kernel():
  mesh = plsc.VectorSubcoreMesh(core_axis_name="c", subcore_axis_name="s")

  @functools.partial(
      pl.kernel,
      out_type=jax.ShapeDtypeStruct((NC, N_NODES, HALF), jnp.float32),
      mesh=mesh,
      scratch_types=[
          pltpu.VMEM((CHUNK,), jnp.int32),
          pltpu.VMEM((CHUNK,), jnp.int32),
          pltpu.VMEM((CHUNK, HALF), jnp.float32),
          pltpu.VMEM_SHARED((ACC_ROWS, HALF), jnp.float32),
          pltpu.SemaphoreType.DMA,
      ],
  )
  def prop_kernel(hs_hbm, src2_hbm, dst_hbm, out_hbm, sbuf, dbuf, rows, acc, sem):
    c = lax.axis_index("c")
    s = lax.axis_index("s")

    zv = jnp.zeros((L,), jnp.float32)

    def zrows(i, _):
        r = i // (HALF // L)
        j = i % (HALF // L)
        rows[r, pl.ds(j * L, L)] = zv
        return 0

    lax.fori_loop(0, CHUNK * (HALF // L), zrows, 0)

    def zacc(k, _):
        pltpu.sync_copy(rows, acc.at[pl.ds(s * ZPT + k * CHUNK, CHUNK)])
        return 0

    lax.fori_loop(0, ZPT // CHUNK, zacc, 0)
    plsc.subcore_barrier()

    def chunk_body(g, _):
        eb = s * EPT + g * CHUNK
        pltpu.sync_copy(src2_hbm.at[c, pl.ds(eb, CHUNK)], sbuf)
        pltpu.sync_copy(dst_hbm.at[pl.ds(eb, CHUNK)], dbuf)
        pltpu.async_copy(hs_hbm.at[sbuf], rows, sem).wait()
        pltpu.sync_copy(rows, acc.at[dbuf], add=True)
        return 0

    lax.fori_loop(0, NCHUNKS, chunk_body, 0)
    plsc.subcore_barrier()

    pltpu.sync_copy(acc.at[pl.ds(s * 624, 624)], out_hbm.at[c, pl.ds(s * 624, 624)])

    @pl.when(s == NS - 1)
    def _():
        pltpu.sync_copy(acc.at[pl.ds(9984, 16)], out_hbm.at[c, pl.ds(9984, 16)])

  return prop_kernel


def _prop_kernel(hs_flat, src2, dst_p):
  return _build_prop_kernel()(hs_flat, src2, dst_p)


def _dinv_from_partials(degp_blk):
    deg = degp_blk[0, :, 0] + degp_blk[1, :, 0] + 1.0
    return lax.rsqrt(deg)


def _mm1_body(x_ref, w_ref, degp_ref, out_ref):
    dinv = _dinv_from_partials(degp_ref[...])
    h = jnp.dot(x_ref[...], w_ref[...], preferred_element_type=jnp.float32)
    hp = h * dinv[:, None]
    out_ref[0] = hp[:, :HALF]
    out_ref[1] = hp[:, HALF:]


def _mm1(x, w1, degp):
    return pl.pallas_call(
        _mm1_body,
        grid=(GRID,),
        in_specs=[
            pl.BlockSpec((ROWBLK, D), lambda i: (i, 0)),
            pl.BlockSpec((D, D), lambda i: (0, 0)),
            pl.BlockSpec((NC, ROWBLK, DEG_W), lambda i: (0, i, 0)),
        ],
        out_specs=pl.BlockSpec((NC, ROWBLK, HALF), lambda i: (0, i, 0)),
        out_shape=jax.ShapeDtypeStruct((NC, N_NODES, HALF), jnp.float32),
    )(x, w1, degp)


def _zstats_body(acc_ref, hs_ref, degp_ref, b_ref, z_ref, stats_ref, s0, s1):
    i = pl.program_id(0)
    dinv = _dinv_from_partials(degp_ref[...])
    accv = jnp.concatenate([acc_ref[0], acc_ref[1]], axis=1)
    hp = jnp.concatenate([hs_ref[0], hs_ref[1]], axis=1)
    z = dinv[:, None] * (accv + hp) + b_ref[...]
    z_ref[...] = z

    @pl.when(i == 0)
    def _():
        s0[...] = jnp.zeros_like(s0)
        s1[...] = jnp.zeros_like(s1)

    s0[...] += jnp.sum(z, axis=0, keepdims=True)
    s1[...] += jnp.sum(z * z, axis=0, keepdims=True)

    @pl.when(i == GRID - 1)
    def _():
        stats_ref[...] = jnp.concatenate([s0[...], s1[...]], axis=0)


def _zstats(acc1, hs1, degp, b1):
    return pl.pallas_call(
        _zstats_body,
        grid=(GRID,),
        in_specs=[
            pl.BlockSpec((NC, ROWBLK, HALF), lambda i: (0, i, 0)),
            pl.BlockSpec((NC, ROWBLK, HALF), lambda i: (0, i, 0)),
            pl.BlockSpec((NC, ROWBLK, DEG_W), lambda i: (0, i, 0)),
            pl.BlockSpec((1, D), lambda i: (0, 0)),
        ],
        out_specs=[
            pl.BlockSpec((ROWBLK, D), lambda i: (i, 0)),
            pl.BlockSpec((2, D), lambda i: (0, 0)),
        ],
        out_shape=[
            jax.ShapeDtypeStruct((N_NODES, D), jnp.float32),
            jax.ShapeDtypeStruct((2, D), jnp.float32),
        ],
        scratch_shapes=[
            pltpu.VMEM((1, D), jnp.float32),
            pltpu.VMEM((1, D), jnp.float32),
        ],
    )(acc1, hs1, degp, b1)


def _layer2_body(z_ref, stats_ref, g_ref, be_ref, w_ref, degp_ref, out_ref):
    mean = stats_ref[0:1, :] * (1.0 / N_NODES)
    ex2 = stats_ref[1:2, :] * (1.0 / N_NODES)
    var = ex2 - mean * mean
    xhat = (z_ref[...] - mean) * lax.rsqrt(var + EPS)
    y = jnp.maximum(xhat * g_ref[...] + be_ref[...], 0.0)
    h2 = jnp.dot(y, w_ref[...], preferred_element_type=jnp.float32)
    dinv = _dinv_from_partials(degp_ref[...])
    hp2 = h2 * dinv[:, None]
    out_ref[0] = hp2[:, :HALF]
    out_ref[1] = hp2[:, HALF:]


def _layer2(z1, stats, gamma, beta, w2, degp):
    return pl.pallas_call(
        _layer2_body,
        grid=(GRID,),
        in_specs=[
            pl.BlockSpec((ROWBLK, D), lambda i: (i, 0)),
            pl.BlockSpec((2, D), lambda i: (0, 0)),
            pl.BlockSpec((1, D), lambda i: (0, 0)),
            pl.BlockSpec((1, D), lambda i: (0, 0)),
            pl.BlockSpec((D, D), lambda i: (0, 0)),
            pl.BlockSpec((NC, ROWBLK, DEG_W), lambda i: (0, i, 0)),
        ],
        out_specs=pl.BlockSpec((NC, ROWBLK, HALF), lambda i: (0, i, 0)),
        out_shape=jax.ShapeDtypeStruct((NC, N_NODES, HALF), jnp.float32),
    )(z1, stats, gamma, beta, w2, degp)


def _final_body(acc_ref, hs_ref, degp_ref, b_ref, o_ref):
    dinv = _dinv_from_partials(degp_ref[...])
    accv = jnp.concatenate([acc_ref[0], acc_ref[1]], axis=1)
    hp = jnp.concatenate([hs_ref[0], hs_ref[1]], axis=1)
    o_ref[...] = dinv[:, None] * (accv + hp) + b_ref[...]


def _final(acc2, hs2, degp, b2):
    return pl.pallas_call(
        _final_body,
        grid=(GRID,),
        in_specs=[
            pl.BlockSpec((NC, ROWBLK, HALF), lambda i: (0, i, 0)),
            pl.BlockSpec((NC, ROWBLK, HALF), lambda i: (0, i, 0)),
            pl.BlockSpec((NC, ROWBLK, DEG_W), lambda i: (0, i, 0)),
            pl.BlockSpec((1, D), lambda i: (0, 0)),
        ],
        out_specs=pl.BlockSpec((ROWBLK, D), lambda i: (i, 0)),
        out_shape=jax.ShapeDtypeStruct((N_NODES, D), jnp.float32),
    )(acc2, hs2, degp, b2)


def kernel(x, edge_index, W1, b1, gamma, beta, W2, b2):
    ei = edge_index.astype(jnp.int32)
    src = ei[0]
    dst = ei[1]
    npad = E_PAD - N_EDGES
    src_p = jnp.concatenate([src, jnp.zeros((npad,), jnp.int32)])
    dst_p = jnp.concatenate([dst, jnp.full((npad,), N_NODES, jnp.int32)])
    src2 = jnp.stack([src_p, src_p + N_NODES])

    b1r = b1.reshape(1, D)
    b2r = b2.reshape(1, D)
    gammar = gamma.reshape(1, D)
    betar = beta.reshape(1, D)

    degp = _deg_kernel(dst_p)
    hs1 = _mm1(x, W1, degp)
    acc1 = _prop_kernel(hs1.reshape(NC * N_NODES, HALF), src2, dst_p)
    z1, stats = _zstats(acc1, hs1, degp, b1r)
    hs2 = _layer2(z1, stats, gammar, betar, W2, degp)
    acc2 = _prop_kernel(hs2.reshape(NC * N_NODES, HALF), src2, dst_p)
    return _final(acc2, hs2, degp, b2r)


# speedup vs baseline: 8.2280x; 1.3280x over previous
import functools

import jax
import jax.numpy as jnp
from jax import lax
from jax.experimental import pallas as pl
from jax.experimental.pallas import tpu as pltpu
from jax.experimental.pallas import tpu_sc as plsc

N_NODES = 10000
D = 256
HALF = 128
N_EDGES = 160000
EPS = 1e-5

NC = 2
NS = 16
L = 16

CHUNK = 128
EPT = 10240
E_PAD = EPT * NS
NCHUNKS = EPT // CHUNK
ACC_ROWS = 10240
ZPT = ACC_ROWS // NS
RPT = N_NODES // NS
DEG_EPW = E_PAD // (NC * NS)

ROWBLK = 1000
GRID = N_NODES // ROWBLK

DEG_W = 16
DEG_CH = DEG_EPW // CHUNK


@functools.cache
def _build_deg_kernel():
  mesh = plsc.VectorSubcoreMesh(core_axis_name="c", subcore_axis_name="s")

  @functools.partial(
      pl.kernel,
      out_type=jax.ShapeDtypeStruct((NC, ACC_ROWS, DEG_W), jnp.float32),
      mesh=mesh,
      scratch_types=[
          pltpu.VMEM((CHUNK,), jnp.int32),
          pltpu.VMEM((CHUNK, DEG_W), jnp.float32),
          pltpu.VMEM((CHUNK, DEG_W), jnp.float32),
          pltpu.VMEM_SHARED((ACC_ROWS, DEG_W), jnp.float32),
      ],
  )
  def deg_kernel(dst_hbm, degp_hbm, dbuf, ones, zbuf, table):
    c = lax.axis_index("c")
    s = lax.axis_index("s")

    onev = jnp.ones((L,), jnp.float32)
    zv = jnp.zeros((L,), jnp.float32)

    def fbody(i, _):
        ones[i, pl.ds(0, L)] = onev
        zbuf[i, pl.ds(0, L)] = zv
        return 0

    lax.fori_loop(0, CHUNK, fbody, 0)

    def zacc(k, _):
        pltpu.sync_copy(zbuf, table.at[pl.ds(s * ZPT + k * CHUNK, CHUNK)])
        return 0

    lax.fori_loop(0, ZPT // CHUNK, zacc, 0)
    plsc.subcore_barrier()

    def body(g, _):
        eb = (c * NS + s) * DEG_EPW + g * CHUNK
        pltpu.sync_copy(dst_hbm.at[pl.ds(eb, CHUNK)], dbuf)
        pltpu.sync_copy(ones, table.at[dbuf], add=True)
        return 0

    lax.fori_loop(0, DEG_CH, body, 0)
    plsc.subcore_barrier()

    pltpu.sync_copy(
        table.at[pl.ds(s * ZPT, ZPT)], degp_hbm.at[c, pl.ds(s * ZPT, ZPT)]
    )

  return deg_kernel


def _deg_kernel(dst_p):
  return _build_deg_kernel()(dst_p)


NBUF = 2
HC = NCHUNKS // 2


@functools.cache
def _build_prop_kernel():
  mesh = plsc.VectorSubcoreMesh(core_axis_name="c", subcore_axis_name="s")

  @functools.partial(
      pl.kernel,
      out_type=jax.ShapeDtypeStruct((NC, N_NODES, HALF), jnp.float32),
      mesh=mesh,
      scratch_types=[
          pltpu.VMEM((HC, CHUNK), jnp.int32),
          pltpu.VMEM((HC, 1, CHUNK), jnp.int32),
          pltpu.VMEM((NBUF, CHUNK, HALF), jnp.float32),
      ]
      + [pltpu.SemaphoreType.DMA] * NBUF
      + [pltpu.SemaphoreType.DMA] * NBUF
      + [pltpu.VMEM_SHARED((ACC_ROWS, HALF), jnp.float32)],
  )
  def prop_kernel(hs_hbm, src2_hbm, dst_hbm, out_hbm, sidx, didx, bufs, *rest):
    gsem = rest[:NBUF]
    ssem = rest[NBUF:2 * NBUF]
    acc = rest[2 * NBUF]
    c = lax.axis_index("c")
    s = lax.axis_index("s")

    zv = jnp.zeros((L,), jnp.float32)
    rows0 = bufs.at[0]

    def zrows(i, _):
        r = i // (HALF // L)
        j = i % (HALF // L)
        rows0[r, pl.ds(j * L, L)] = zv
        return 0

    lax.fori_loop(0, CHUNK * (HALF // L), zrows, 0)

    def zacc(k, _):
        pltpu.sync_copy(rows0, acc.at[pl.ds(s * ZPT + k * CHUNK, CHUNK)])
        return 0

    lax.fori_loop(0, ZPT // CHUNK, zacc, 0)
    plsc.subcore_barrier()

    def _wait_gather(b):
        pltpu.make_async_copy(hs_hbm.at[sidx.at[b]], bufs.at[b], gsem[b]).wait()

    def _wait_scatter(b):
        pltpu.make_async_copy(bufs.at[b], acc.at[didx.at[b, 0]], ssem[b]).wait()

    for h in range(2):
        pltpu.sync_copy(src2_hbm.at[c, s, pl.ds(h * HC, HC)], sidx)
        pltpu.sync_copy(dst_hbm.at[s, pl.ds(h * HC, HC)], didx)

        for b in range(NBUF):
            pltpu.async_copy(hs_hbm.at[sidx.at[b]], bufs.at[b], gsem[b])

        def tbody(t, _):
            for b in range(NBUF):
                g = t * NBUF + b
                _wait_gather(b)
                pltpu.async_copy(bufs.at[b], acc.at[didx.at[g, 0]], ssem[b], add=True)

                @pl.when(g + NBUF < HC)
                def _():
                    _wait_scatter(b)
                    pltpu.async_copy(hs_hbm.at[sidx.at[g + NBUF]], bufs.at[b], gsem[b])

            return 0

        lax.fori_loop(0, HC // NBUF, tbody, 0)
        for b in range(NBUF):
            _wait_scatter(b)
    plsc.subcore_barrier()

    pltpu.sync_copy(acc.at[pl.ds(s * 624, 624)], out_hbm.at[c, pl.ds(s * 624, 624)])

    @pl.when(s == NS - 1)
    def _():
        pltpu.sync_copy(acc.at[pl.ds(9984, 16)], out_hbm.at[c, pl.ds(9984, 16)])

  return prop_kernel


def _prop_kernel(hs_flat, src2, dst_p):
  return _build_prop_kernel()(hs_flat, src2, dst_p)


def _dinv_from_partials(degp_blk):
    deg = degp_blk[0, :, 0] + degp_blk[1, :, 0] + 1.0
    return lax.rsqrt(deg)


def _mm1_body(x_ref, w_ref, degp_ref, out_ref):
    dinv = _dinv_from_partials(degp_ref[...])
    h = jnp.dot(x_ref[...], w_ref[...], preferred_element_type=jnp.float32)
    hp = h * dinv[:, None]
    out_ref[0] = hp[:, :HALF]
    out_ref[1] = hp[:, HALF:]


def _mm1(x, w1, degp):
    return pl.pallas_call(
        _mm1_body,
        grid=(GRID,),
        in_specs=[
            pl.BlockSpec((ROWBLK, D), lambda i: (i, 0)),
            pl.BlockSpec((D, D), lambda i: (0, 0)),
            pl.BlockSpec((NC, ROWBLK, DEG_W), lambda i: (0, i, 0)),
        ],
        out_specs=pl.BlockSpec((NC, ROWBLK, HALF), lambda i: (0, i, 0)),
        out_shape=jax.ShapeDtypeStruct((NC, N_NODES, HALF), jnp.float32),
    )(x, w1, degp)


def _zstats_body(acc_ref, hs_ref, degp_ref, b_ref, z_ref, stats_ref, s0, s1):
    i = pl.program_id(0)
    dinv = _dinv_from_partials(degp_ref[...])
    accv = jnp.concatenate([acc_ref[0], acc_ref[1]], axis=1)
    hp = jnp.concatenate([hs_ref[0], hs_ref[1]], axis=1)
    z = dinv[:, None] * (accv + hp) + b_ref[...]
    z_ref[...] = z

    @pl.when(i == 0)
    def _():
        s0[...] = jnp.zeros_like(s0)
        s1[...] = jnp.zeros_like(s1)

    s0[...] += jnp.sum(z, axis=0, keepdims=True)
    s1[...] += jnp.sum(z * z, axis=0, keepdims=True)

    @pl.when(i == GRID - 1)
    def _():
        stats_ref[...] = jnp.concatenate([s0[...], s1[...]], axis=0)


def _zstats(acc1, hs1, degp, b1):
    return pl.pallas_call(
        _zstats_body,
        grid=(GRID,),
        in_specs=[
            pl.BlockSpec((NC, ROWBLK, HALF), lambda i: (0, i, 0)),
            pl.BlockSpec((NC, ROWBLK, HALF), lambda i: (0, i, 0)),
            pl.BlockSpec((NC, ROWBLK, DEG_W), lambda i: (0, i, 0)),
            pl.BlockSpec((1, D), lambda i: (0, 0)),
        ],
        out_specs=[
            pl.BlockSpec((ROWBLK, D), lambda i: (i, 0)),
            pl.BlockSpec((2, D), lambda i: (0, 0)),
        ],
        out_shape=[
            jax.ShapeDtypeStruct((N_NODES, D), jnp.float32),
            jax.ShapeDtypeStruct((2, D), jnp.float32),
        ],
        scratch_shapes=[
            pltpu.VMEM((1, D), jnp.float32),
            pltpu.VMEM((1, D), jnp.float32),
        ],
    )(acc1, hs1, degp, b1)


def _layer2_body(z_ref, stats_ref, g_ref, be_ref, w_ref, degp_ref, out_ref):
    mean = stats_ref[0:1, :] * (1.0 / N_NODES)
    ex2 = stats_ref[1:2, :] * (1.0 / N_NODES)
    var = ex2 - mean * mean
    xhat = (z_ref[...] - mean) * lax.rsqrt(var + EPS)
    y = jnp.maximum(xhat * g_ref[...] + be_ref[...], 0.0)
    h2 = jnp.dot(y, w_ref[...], preferred_element_type=jnp.float32)
    dinv = _dinv_from_partials(degp_ref[...])
    hp2 = h2 * dinv[:, None]
    out_ref[0] = hp2[:, :HALF]
    out_ref[1] = hp2[:, HALF:]


def _layer2(z1, stats, gamma, beta, w2, degp):
    return pl.pallas_call(
        _layer2_body,
        grid=(GRID,),
        in_specs=[
            pl.BlockSpec((ROWBLK, D), lambda i: (i, 0)),
            pl.BlockSpec((2, D), lambda i: (0, 0)),
            pl.BlockSpec((1, D), lambda i: (0, 0)),
            pl.BlockSpec((1, D), lambda i: (0, 0)),
            pl.BlockSpec((D, D), lambda i: (0, 0)),
            pl.BlockSpec((NC, ROWBLK, DEG_W), lambda i: (0, i, 0)),
        ],
        out_specs=pl.BlockSpec((NC, ROWBLK, HALF), lambda i: (0, i, 0)),
        out_shape=jax.ShapeDtypeStruct((NC, N_NODES, HALF), jnp.float32),
    )(z1, stats, gamma, beta, w2, degp)


def _final_body(acc_ref, hs_ref, degp_ref, b_ref, o_ref):
    dinv = _dinv_from_partials(degp_ref[...])
    accv = jnp.concatenate([acc_ref[0], acc_ref[1]], axis=1)
    hp = jnp.concatenate([hs_ref[0], hs_ref[1]], axis=1)
    o_ref[...] = dinv[:, None] * (accv + hp) + b_ref[...]


def _final(acc2, hs2, degp, b2):
    return pl.pallas_call(
        _final_body,
        grid=(GRID,),
        in_specs=[
            pl.BlockSpec((NC, ROWBLK, HALF), lambda i: (0, i, 0)),
            pl.BlockSpec((NC, ROWBLK, HALF), lambda i: (0, i, 0)),
            pl.BlockSpec((NC, ROWBLK, DEG_W), lambda i: (0, i, 0)),
            pl.BlockSpec((1, D), lambda i: (0, 0)),
        ],
        out_specs=pl.BlockSpec((ROWBLK, D), lambda i: (i, 0)),
        out_shape=jax.ShapeDtypeStruct((N_NODES, D), jnp.float32),
    )(acc2, hs2, degp, b2)


def kernel(x, edge_index, W1, b1, gamma, beta, W2, b2):
    ei = edge_index.astype(jnp.int32)
    src = ei[0]
    dst = ei[1]
    npad = E_PAD - N_EDGES
    src_p = jnp.concatenate([src, jnp.zeros((npad,), jnp.int32)])
    dst_p = jnp.concatenate([dst, jnp.full((npad,), N_NODES, jnp.int32)])
    src2 = jnp.stack([src_p, src_p + N_NODES]).reshape(NC, NS, NCHUNKS, CHUNK)
    dst_r = dst_p.reshape(NS, NCHUNKS, 1, CHUNK)

    b1r = b1.reshape(1, D)
    b2r = b2.reshape(1, D)
    gammar = gamma.reshape(1, D)
    betar = beta.reshape(1, D)

    degp = _deg_kernel(dst_p)
    hs1 = _mm1(x, W1, degp)
    acc1 = _prop_kernel(hs1.reshape(NC * N_NODES, HALF), src2, dst_r)
    z1, stats = _zstats(acc1, hs1, degp, b1r)
    hs2 = _layer2(z1, stats, gammar, betar, W2, degp)
    acc2 = _prop_kernel(hs2.reshape(NC * N_NODES, HALF), src2, dst_r)
    return _final(acc2, hs2, degp, b2r)


# speedup vs baseline: 8.4385x; 1.0256x over previous
import functools

import jax
import jax.numpy as jnp
from jax import lax
from jax.experimental import pallas as pl
from jax.experimental.pallas import tpu as pltpu
from jax.experimental.pallas import tpu_sc as plsc

N_NODES = 10000
D = 256
HALF = 128
N_EDGES = 160000
EPS = 1e-5

NC = 2
NS = 16
L = 16

CHUNK = 64
EPT = 10240
E_PAD = EPT * NS
NCHUNKS = EPT // CHUNK
ACC_ROWS = 10240
ZPT = ACC_ROWS // NS
RPT = N_NODES // NS
DEG_EPW = E_PAD // (NC * NS)

ROWBLK = 1000
GRID = N_NODES // ROWBLK

DEG_W = 16
DEG_CH = DEG_EPW // CHUNK


@functools.cache
def _build_deg_kernel():
  mesh = plsc.VectorSubcoreMesh(core_axis_name="c", subcore_axis_name="s")

  @functools.partial(
      pl.kernel,
      out_type=jax.ShapeDtypeStruct((NC, ACC_ROWS, DEG_W), jnp.float32),
      mesh=mesh,
      scratch_types=[
          pltpu.VMEM((CHUNK,), jnp.int32),
          pltpu.VMEM((CHUNK, DEG_W), jnp.float32),
          pltpu.VMEM((CHUNK, DEG_W), jnp.float32),
          pltpu.VMEM_SHARED((ACC_ROWS, DEG_W), jnp.float32),
      ],
  )
  def deg_kernel(dst_hbm, degp_hbm, dbuf, ones, zbuf, table):
    c = lax.axis_index("c")
    s = lax.axis_index("s")

    onev = jnp.ones((L,), jnp.float32)
    zv = jnp.zeros((L,), jnp.float32)

    def fbody(i, _):
        ones[i, pl.ds(0, L)] = onev
        zbuf[i, pl.ds(0, L)] = zv
        return 0

    lax.fori_loop(0, CHUNK, fbody, 0)

    def zacc(k, _):
        pltpu.sync_copy(zbuf, table.at[pl.ds(s * ZPT + k * CHUNK, CHUNK)])
        return 0

    lax.fori_loop(0, ZPT // CHUNK, zacc, 0)
    plsc.subcore_barrier()

    def body(g, _):
        eb = (c * NS + s) * DEG_EPW + g * CHUNK
        pltpu.sync_copy(dst_hbm.at[pl.ds(eb, CHUNK)], dbuf)
        pltpu.sync_copy(ones, table.at[dbuf], add=True)
        return 0

    lax.fori_loop(0, DEG_CH, body, 0)
    plsc.subcore_barrier()

    pltpu.sync_copy(
        table.at[pl.ds(s * ZPT, ZPT)], degp_hbm.at[c, pl.ds(s * ZPT, ZPT)]
    )

  return deg_kernel


def _deg_kernel(dst_p):
  return _build_deg_kernel()(dst_p)


NBUF = 4
NSTAGE = 4
HC = NCHUNKS // NSTAGE


@functools.cache
def _build_prop_kernel():
  mesh = plsc.VectorSubcoreMesh(core_axis_name="c", subcore_axis_name="s")

  @functools.partial(
      pl.kernel,
      out_type=jax.ShapeDtypeStruct((NC, N_NODES, HALF), jnp.float32),
      mesh=mesh,
      scratch_types=[
          pltpu.VMEM((HC, CHUNK), jnp.int32),
          pltpu.VMEM((HC, 1, CHUNK), jnp.int32),
          pltpu.VMEM((NBUF, CHUNK, HALF), jnp.float32),
      ]
      + [pltpu.SemaphoreType.DMA] * NBUF
      + [pltpu.SemaphoreType.DMA] * NBUF
      + [pltpu.VMEM_SHARED((ACC_ROWS, HALF), jnp.float32)],
  )
  def prop_kernel(hs_hbm, src2_hbm, dst_hbm, out_hbm, sidx, didx, bufs, *rest):
    gsem = rest[:NBUF]
    ssem = rest[NBUF:2 * NBUF]
    acc = rest[2 * NBUF]
    c = lax.axis_index("c")
    s = lax.axis_index("s")

    zv = jnp.zeros((L,), jnp.float32)
    rows0 = bufs.at[0]

    def zrows(i, _):
        r = i // (HALF // L)
        j = i % (HALF // L)
        rows0[r, pl.ds(j * L, L)] = zv
        return 0

    lax.fori_loop(0, CHUNK * (HALF // L), zrows, 0)

    def zacc(k, _):
        pltpu.sync_copy(rows0, acc.at[pl.ds(s * ZPT + k * CHUNK, CHUNK)])
        return 0

    lax.fori_loop(0, ZPT // CHUNK, zacc, 0)
    plsc.subcore_barrier()

    def _wait_gather(b):
        pltpu.make_async_copy(hs_hbm.at[sidx.at[b]], bufs.at[b], gsem[b]).wait()

    def _wait_scatter(b):
        pltpu.make_async_copy(bufs.at[b], acc.at[didx.at[b, 0]], ssem[b]).wait()

    for h in range(NSTAGE):
        pltpu.sync_copy(src2_hbm.at[c, s, pl.ds(h * HC, HC)], sidx)
        pltpu.sync_copy(dst_hbm.at[s, pl.ds(h * HC, HC)], didx)

        for b in range(NBUF):
            pltpu.async_copy(hs_hbm.at[sidx.at[b]], bufs.at[b], gsem[b])

        def tbody(t, _):
            for b in range(NBUF):
                g = t * NBUF + b
                _wait_gather(b)
                pltpu.async_copy(bufs.at[b], acc.at[didx.at[g, 0]], ssem[b], add=True)

                @pl.when(g + NBUF < HC)
                def _():
                    _wait_scatter(b)
                    pltpu.async_copy(hs_hbm.at[sidx.at[g + NBUF]], bufs.at[b], gsem[b])

            return 0

        lax.fori_loop(0, HC // NBUF, tbody, 0)
        for b in range(NBUF):
            _wait_scatter(b)
    plsc.subcore_barrier()

    pltpu.sync_copy(acc.at[pl.ds(s * 624, 624)], out_hbm.at[c, pl.ds(s * 624, 624)])

    @pl.when(s == NS - 1)
    def _():
        pltpu.sync_copy(acc.at[pl.ds(9984, 16)], out_hbm.at[c, pl.ds(9984, 16)])

  return prop_kernel


def _prop_kernel(hs_flat, src2, dst_p):
  return _build_prop_kernel()(hs_flat, src2, dst_p)


def _dinv_from_partials(degp_blk):
    deg = degp_blk[0, :, 0] + degp_blk[1, :, 0] + 1.0
    return lax.rsqrt(deg)


def _mm1_body(x_ref, w_ref, degp_ref, out_ref):
    dinv = _dinv_from_partials(degp_ref[...])
    h = jnp.dot(x_ref[...], w_ref[...], preferred_element_type=jnp.float32)
    hp = h * dinv[:, None]
    out_ref[0] = hp[:, :HALF]
    out_ref[1] = hp[:, HALF:]


def _mm1(x, w1, degp):
    return pl.pallas_call(
        _mm1_body,
        grid=(GRID,),
        in_specs=[
            pl.BlockSpec((ROWBLK, D), lambda i: (i, 0)),
            pl.BlockSpec((D, D), lambda i: (0, 0)),
            pl.BlockSpec((NC, ROWBLK, DEG_W), lambda i: (0, i, 0)),
        ],
        out_specs=pl.BlockSpec((NC, ROWBLK, HALF), lambda i: (0, i, 0)),
        out_shape=jax.ShapeDtypeStruct((NC, N_NODES, HALF), jnp.float32),
    )(x, w1, degp)


def _zstats_body(acc_ref, hs_ref, degp_ref, b_ref, z_ref, stats_ref, s0, s1):
    i = pl.program_id(0)
    dinv = _dinv_from_partials(degp_ref[...])
    accv = jnp.concatenate([acc_ref[0], acc_ref[1]], axis=1)
    hp = jnp.concatenate([hs_ref[0], hs_ref[1]], axis=1)
    z = dinv[:, None] * (accv + hp) + b_ref[...]
    z_ref[...] = z

    @pl.when(i == 0)
    def _():
        s0[...] = jnp.zeros_like(s0)
        s1[...] = jnp.zeros_like(s1)

    s0[...] += jnp.sum(z, axis=0, keepdims=True)
    s1[...] += jnp.sum(z * z, axis=0, keepdims=True)

    @pl.when(i == GRID - 1)
    def _():
        stats_ref[...] = jnp.concatenate([s0[...], s1[...]], axis=0)


def _zstats(acc1, hs1, degp, b1):
    return pl.pallas_call(
        _zstats_body,
        grid=(GRID,),
        in_specs=[
            pl.BlockSpec((NC, ROWBLK, HALF), lambda i: (0, i, 0)),
            pl.BlockSpec((NC, ROWBLK, HALF), lambda i: (0, i, 0)),
            pl.BlockSpec((NC, ROWBLK, DEG_W), lambda i: (0, i, 0)),
            pl.BlockSpec((1, D), lambda i: (0, 0)),
        ],
        out_specs=[
            pl.BlockSpec((ROWBLK, D), lambda i: (i, 0)),
            pl.BlockSpec((2, D), lambda i: (0, 0)),
        ],
        out_shape=[
            jax.ShapeDtypeStruct((N_NODES, D), jnp.float32),
            jax.ShapeDtypeStruct((2, D), jnp.float32),
        ],
        scratch_shapes=[
            pltpu.VMEM((1, D), jnp.float32),
            pltpu.VMEM((1, D), jnp.float32),
        ],
    )(acc1, hs1, degp, b1)


def _layer2_body(z_ref, stats_ref, g_ref, be_ref, w_ref, degp_ref, out_ref):
    mean = stats_ref[0:1, :] * (1.0 / N_NODES)
    ex2 = stats_ref[1:2, :] * (1.0 / N_NODES)
    var = ex2 - mean * mean
    xhat = (z_ref[...] - mean) * lax.rsqrt(var + EPS)
    y = jnp.maximum(xhat * g_ref[...] + be_ref[...], 0.0)
    h2 = jnp.dot(y, w_ref[...], preferred_element_type=jnp.float32)
    dinv = _dinv_from_partials(degp_ref[...])
    hp2 = h2 * dinv[:, None]
    out_ref[0] = hp2[:, :HALF]
    out_ref[1] = hp2[:, HALF:]


def _layer2(z1, stats, gamma, beta, w2, degp):
    return pl.pallas_call(
        _layer2_body,
        grid=(GRID,),
        in_specs=[
            pl.BlockSpec((ROWBLK, D), lambda i: (i, 0)),
            pl.BlockSpec((2, D), lambda i: (0, 0)),
            pl.BlockSpec((1, D), lambda i: (0, 0)),
            pl.BlockSpec((1, D), lambda i: (0, 0)),
            pl.BlockSpec((D, D), lambda i: (0, 0)),
            pl.BlockSpec((NC, ROWBLK, DEG_W), lambda i: (0, i, 0)),
        ],
        out_specs=pl.BlockSpec((NC, ROWBLK, HALF), lambda i: (0, i, 0)),
        out_shape=jax.ShapeDtypeStruct((NC, N_NODES, HALF), jnp.float32),
    )(z1, stats, gamma, beta, w2, degp)


def _final_body(acc_ref, hs_ref, degp_ref, b_ref, o_ref):
    dinv = _dinv_from_partials(degp_ref[...])
    accv = jnp.concatenate([acc_ref[0], acc_ref[1]], axis=1)
    hp = jnp.concatenate([hs_ref[0], hs_ref[1]], axis=1)
    o_ref[...] = dinv[:, None] * (accv + hp) + b_ref[...]


def _final(acc2, hs2, degp, b2):
    return pl.pallas_call(
        _final_body,
        grid=(GRID,),
        in_specs=[
            pl.BlockSpec((NC, ROWBLK, HALF), lambda i: (0, i, 0)),
            pl.BlockSpec((NC, ROWBLK, HALF), lambda i: (0, i, 0)),
            pl.BlockSpec((NC, ROWBLK, DEG_W), lambda i: (0, i, 0)),
            pl.BlockSpec((1, D), lambda i: (0, 0)),
        ],
        out_specs=pl.BlockSpec((ROWBLK, D), lambda i: (i, 0)),
        out_shape=jax.ShapeDtypeStruct((N_NODES, D), jnp.float32),
    )(acc2, hs2, degp, b2)


def kernel(x, edge_index, W1, b1, gamma, beta, W2, b2):
    ei = edge_index.astype(jnp.int32)
    src = ei[0]
    dst = ei[1]
    npad = E_PAD - N_EDGES
    src_p = jnp.concatenate([src, jnp.zeros((npad,), jnp.int32)])
    dst_p = jnp.concatenate([dst, jnp.full((npad,), N_NODES, jnp.int32)])
    src2 = jnp.stack([src_p, src_p + N_NODES]).reshape(NC, NS, NCHUNKS, CHUNK)
    dst_r = dst_p.reshape(NS, NCHUNKS, 1, CHUNK)

    b1r = b1.reshape(1, D)
    b2r = b2.reshape(1, D)
    gammar = gamma.reshape(1, D)
    betar = beta.reshape(1, D)

    degp = _deg_kernel(dst_p)
    hs1 = _mm1(x, W1, degp)
    acc1 = _prop_kernel(hs1.reshape(NC * N_NODES, HALF), src2, dst_r)
    z1, stats = _zstats(acc1, hs1, degp, b1r)
    hs2 = _layer2(z1, stats, gammar, betar, W2, degp)
    acc2 = _prop_kernel(hs2.reshape(NC * N_NODES, HALF), src2, dst_r)
    return _final(acc2, hs2, degp, b2r)
